# baseline (speedup 1.0000x reference)
import functools

import jax
import jax.numpy as jnp
from jax import lax
from jax.experimental import pallas as pl
from jax.experimental.pallas import tpu as pltpu
from jax.experimental.pallas import tpu_sc as plsc

K = 16
CH = 32
M = 16
COUT = 64
P = 512
CHUNK = 1024


def _leaky(x):
    return jnp.where(x >= 0, x, 0.1 * x)


def _make_sc_gather(rows):
    info = plsc.get_sparse_core_info()
    nc, ns = info.num_cores, info.num_subcores
    nw = nc * ns
    rows_per_w = rows // nw
    iters = rows_per_w // CHUNK
    mesh = plsc.VectorSubcoreMesh(core_axis_name="c", subcore_axis_name="s")

    @functools.partial(
        pl.kernel, mesh=mesh,
        compiler_params=pltpu.CompilerParams(use_tc_tiling_on_sc=False),
        out_type=jax.ShapeDtypeStruct((rows, CH), jnp.bfloat16),
        scratch_types=[
            pltpu.VMEM((CHUNK,), jnp.int32),
            pltpu.VMEM((CHUNK, CH), jnp.bfloat16),
            pltpu.SemaphoreType.DMA,
        ],
    )
    def sc_gather(table_hbm, idx_hbm, out_hbm, idx_v, rows_v, sem):
        wid = lax.axis_index("s") * nc + lax.axis_index("c")
        base = wid * rows_per_w

        def body(i, carry):
            off = base + i * CHUNK
            pltpu.sync_copy(idx_hbm.at[pl.ds(off, CHUNK)], idx_v)
            pltpu.async_copy(table_hbm.at[idx_v], rows_v, sem).wait()
            pltpu.sync_copy(rows_v, out_hbm.at[pl.ds(off, CHUNK)])
            return carry

        lax.fori_loop(0, iters, body, 0)

    return sc_gather


def _tc_body(g_ref, xyz_ref, bd1_ref, b1_ref, bd2_ref, b2_ref, wl_ref,
             blin_ref, out_ref):
    g = g_ref[...].astype(jnp.float32)
    xyzb = xyz_ref[0]
    x48 = jnp.concatenate(
        [g[j * CH:j * CH + 3, :] for j in range(K)], axis=0)
    xyzrep = jnp.concatenate([xyzb] * K, axis=0)
    dx = x48 - xyzrep
    h = _leaky(jnp.dot(bd1_ref[...], dx,
                       preferred_element_type=jnp.float32) + b1_ref[...])
    w = _leaky(jnp.dot(bd2_ref[...], h,
                       preferred_element_type=jnp.float32) + b2_ref[...])
    accs = []
    for m in range(M):
        am = jnp.zeros((CH, P), dtype=jnp.float32)
        for j in range(K):
            row = w[j * M + m:j * M + m + 1, :]
            am = am + jnp.broadcast_to(row, (CH, P)) * g[j * CH:(j + 1) * CH, :]
        accs.append(am)
    acc = jnp.concatenate(accs, axis=0)
    out = jnp.dot(wl_ref[...], acc, preferred_element_type=jnp.float32)
    out_ref[0] = _leaky(out + blin_ref[...])


def kernel(xyz, features, knn_indices, w1, b1, w2, b2, Wlin, blin):
    B, _, N = xyz.shape
    Cin = features.shape[1]

    xyz_t = jnp.transpose(xyz, (0, 2, 1)).reshape(B * N, 3)
    feat_t = jnp.transpose(features, (0, 2, 1)).reshape(B * N, Cin)
    table = jnp.concatenate(
        [xyz_t, feat_t, jnp.zeros((B * N, CH - 3 - Cin), jnp.float32)],
        axis=1).astype(jnp.bfloat16)

    idx = (knn_indices.astype(jnp.int32)
           + (jnp.arange(B, dtype=jnp.int32) * N)[:, None, None])
    idx = idx.reshape(B * N * K)

    gathered = _make_sc_gather(B * N * K)(table, idx)
    gt = jnp.transpose(gathered.reshape(B * N, K * CH))

    bd1 = jnp.kron(jnp.eye(K, dtype=jnp.float32), w1)
    bd2 = jnp.kron(jnp.eye(K, dtype=jnp.float32), w2)
    b1r = jnp.tile(b1, K).reshape(8 * K, 1)
    b2r = jnp.tile(b2, K).reshape(M * K, 1)

    wl = Wlin.reshape(COUT, M, 19)
    wl = jnp.concatenate(
        [wl, jnp.zeros((COUT, M, CH - 19), jnp.float32)], axis=2)
    wl = wl.reshape(COUT, M * CH)

    nb = N // P
    out = pl.pallas_call(
        _tc_body,
        grid=(B, nb),
        in_specs=[
            pl.BlockSpec((K * CH, P), lambda b, i: (0, b * nb + i)),
            pl.BlockSpec((1, 3, P), lambda b, i: (b, 0, i)),
            pl.BlockSpec((8 * K, 3 * K), lambda b, i: (0, 0)),
            pl.BlockSpec((8 * K, 1), lambda b, i: (0, 0)),
            pl.BlockSpec((M * K, 8 * K), lambda b, i: (0, 0)),
            pl.BlockSpec((M * K, 1), lambda b, i: (0, 0)),
            pl.BlockSpec((COUT, M * CH), lambda b, i: (0, 0)),
            pl.BlockSpec((COUT, 1), lambda b, i: (0, 0)),
        ],
        out_specs=pl.BlockSpec((1, COUT, P), lambda b, i: (b, 0, i)),
        out_shape=jax.ShapeDtypeStruct((B, COUT, N), jnp.float32),
    )(gt, xyz, bd1, b1r, bd2, b2r, wl, blin.reshape(COUT, 1))

    return out


# speedup vs baseline: 1.1691x; 1.1691x over previous
import functools

import jax
import jax.numpy as jnp
from jax import lax
from jax.experimental import pallas as pl
from jax.experimental.pallas import tpu as pltpu
from jax.experimental.pallas import tpu_sc as plsc

K = 16
CH = 32
M = 16
CT = 24
COUT = 64
P = 512
CHUNK = 1024


def _leaky(x):
    return jnp.where(x >= 0, x, 0.1 * x)


def _make_sc_gather(rows):
    info = plsc.get_sparse_core_info()
    nc, ns = info.num_cores, info.num_subcores
    nw = nc * ns
    rows_per_w = rows // nw
    iters = rows_per_w // CHUNK
    mesh = plsc.VectorSubcoreMesh(core_axis_name="c", subcore_axis_name="s")

    @functools.partial(
        pl.kernel, mesh=mesh,
        compiler_params=pltpu.CompilerParams(use_tc_tiling_on_sc=False),
        out_type=jax.ShapeDtypeStruct((rows, CH), jnp.float32),
        scratch_types=[
            pltpu.VMEM((CHUNK,), jnp.int32),
            pltpu.VMEM((CHUNK, CH), jnp.float32),
            pltpu.SemaphoreType.DMA,
        ],
    )
    def sc_gather(table_hbm, idx_hbm, out_hbm, idx_v, rows_v, sem):
        wid = lax.axis_index("s") * nc + lax.axis_index("c")
        base = wid * rows_per_w

        def body(i, carry):
            off = base + i * CHUNK
            pltpu.sync_copy(idx_hbm.at[pl.ds(off, CHUNK)], idx_v)
            pltpu.async_copy(table_hbm.at[idx_v], rows_v, sem).wait()
            pltpu.sync_copy(rows_v, out_hbm.at[pl.ds(off, CHUNK)])
            return carry

        lax.fori_loop(0, iters, body, 0)

    return sc_gather


def _tc_body(g_ref, xyz_ref, bd1_ref, b1_ref, bd2_ref, b2_ref, wl_ref,
             blin_ref, out_ref):
    xyzb = xyz_ref[0]
    x48 = jnp.concatenate(
        [g_ref[j * CH:j * CH + 3, :] for j in range(K)], axis=0)
    xyzrep = jnp.concatenate([xyzb] * K, axis=0)
    dx = (x48 - xyzrep).astype(jnp.bfloat16)
    h = _leaky(jnp.dot(bd1_ref[...], dx,
                       preferred_element_type=jnp.float32) + b1_ref[...])
    w = _leaky(jnp.dot(bd2_ref[...], h.astype(jnp.bfloat16),
                       preferred_element_type=jnp.float32) + b2_ref[...])
    accs = []
    for m in range(M):
        am = jnp.zeros((CT, P), dtype=jnp.float32)
        for j in range(K):
            row = w[j * M + m:j * M + m + 1, :]
            am = am + jnp.broadcast_to(row, (CT, P)) * g_ref[j * CH:j * CH + CT, :]
        accs.append(am)
    acc = jnp.concatenate(accs, axis=0)
    out = jnp.dot(wl_ref[...], acc.astype(jnp.bfloat16),
                  preferred_element_type=jnp.float32)
    out_ref[0] = _leaky(out + blin_ref[...])


def kernel(xyz, features, knn_indices, w1, b1, w2, b2, Wlin, blin):
    B, _, N = xyz.shape
    Cin = features.shape[1]

    xyz_t = jnp.transpose(xyz, (0, 2, 1)).reshape(B * N, 3)
    feat_t = jnp.transpose(features, (0, 2, 1)).reshape(B * N, Cin)
    table = jnp.concatenate(
        [xyz_t, feat_t, jnp.zeros((B * N, CH - 3 - Cin), jnp.float32)], axis=1)

    idx = (knn_indices.astype(jnp.int32)
           + (jnp.arange(B, dtype=jnp.int32) * N)[:, None, None])
    idx = idx.reshape(B * N * K)

    gathered = _make_sc_gather(B * N * K)(table, idx)
    gt = jnp.transpose(gathered.reshape(B * N, K * CH))

    bd1 = jnp.kron(jnp.eye(K, dtype=jnp.float32), w1).astype(jnp.bfloat16)
    bd2 = jnp.kron(jnp.eye(K, dtype=jnp.float32), w2).astype(jnp.bfloat16)
    b1r = jnp.tile(b1, K).reshape(8 * K, 1)
    b2r = jnp.tile(b2, K).reshape(M * K, 1)

    wl = Wlin.reshape(COUT, M, 19)
    wl = jnp.concatenate(
        [wl, jnp.zeros((COUT, M, CT - 19), jnp.float32)], axis=2)
    wl = wl.reshape(COUT, M * CT).astype(jnp.bfloat16)

    nb = N // P
    out = pl.pallas_call(
        _tc_body,
        grid=(B, nb),
        in_specs=[
            pl.BlockSpec((K * CH, P), lambda b, i: (0, b * nb + i)),
            pl.BlockSpec((1, 3, P), lambda b, i: (b, 0, i)),
            pl.BlockSpec((8 * K, 3 * K), lambda b, i: (0, 0)),
            pl.BlockSpec((8 * K, 1), lambda b, i: (0, 0)),
            pl.BlockSpec((M * K, 8 * K), lambda b, i: (0, 0)),
            pl.BlockSpec((M * K, 1), lambda b, i: (0, 0)),
            pl.BlockSpec((COUT, M * CT), lambda b, i: (0, 0)),
            pl.BlockSpec((COUT, 1), lambda b, i: (0, 0)),
        ],
        out_specs=pl.BlockSpec((1, COUT, P), lambda b, i: (b, 0, i)),
        out_shape=jax.ShapeDtypeStruct((B, COUT, N), jnp.float32),
    )(gt, xyz, bd1, b1r, bd2, b2r, wl, blin.reshape(COUT, 1))

    return out


# speedup vs baseline: 1.2079x; 1.0332x over previous
import functools

import jax
import jax.numpy as jnp
from jax import lax
from jax.experimental import pallas as pl
from jax.experimental.pallas import tpu as pltpu
from jax.experimental.pallas import tpu_sc as plsc

K = 16
CH = 32
M = 16
CT = 24
COUT = 64
P = 512
CHUNK = 512


def _leaky(x):
    return jnp.where(x >= 0, x, 0.1 * x)


def _make_sc_gather(rows):
    info = plsc.get_sparse_core_info()
    nc, ns = info.num_cores, info.num_subcores
    nw = nc * ns
    rows_per_w = rows // nw
    n2 = rows_per_w // (2 * CHUNK)
    mesh = plsc.VectorSubcoreMesh(core_axis_name="c", subcore_axis_name="s")

    @functools.partial(
        pl.kernel, mesh=mesh,
        compiler_params=pltpu.CompilerParams(use_tc_tiling_on_sc=False),
        out_type=jax.ShapeDtypeStruct((rows, CH), jnp.float32),
        scratch_types=[
            pltpu.VMEM((rows_per_w,), jnp.int32),
            pltpu.VMEM((CHUNK, CH), jnp.float32),
            pltpu.VMEM((CHUNK, CH), jnp.float32),
            pltpu.SemaphoreType.DMA,
            pltpu.SemaphoreType.DMA,
        ],
    )
    def sc_gather(table_hbm, idx_hbm, out_hbm, idx_v, rows_a, rows_b,
                  sem_a, sem_b):
        wid = lax.axis_index("s") * nc + lax.axis_index("c")
        base = wid * rows_per_w
        pltpu.sync_copy(idx_hbm.at[pl.ds(base, rows_per_w)], idx_v)

        def body(i, carry):
            c0 = 2 * i * CHUNK
            c1 = c0 + CHUNK
            ha = pltpu.async_copy(
                table_hbm.at[idx_v.at[pl.ds(c0, CHUNK)]], rows_a, sem_a)
            hb = pltpu.async_copy(
                table_hbm.at[idx_v.at[pl.ds(c1, CHUNK)]], rows_b, sem_b)
            ha.wait()
            pltpu.sync_copy(rows_a, out_hbm.at[pl.ds(base + c0, CHUNK)])
            hb.wait()
            pltpu.sync_copy(rows_b, out_hbm.at[pl.ds(base + c1, CHUNK)])
            return carry

        lax.fori_loop(0, n2, body, 0)

    return sc_gather


def _tc_body(g_ref, xyz_ref, bd1_ref, b1_ref, bd2_ref, b2_ref, wl_ref,
             blin_ref, out_ref):
    xyzb = xyz_ref[0]
    x48 = jnp.concatenate(
        [g_ref[j * CH:j * CH + 3, :] for j in range(K)], axis=0)
    xyzrep = jnp.concatenate([xyzb] * K, axis=0)
    dx = (x48 - xyzrep).astype(jnp.bfloat16)
    h = _leaky(jnp.dot(bd1_ref[...], dx,
                       preferred_element_type=jnp.float32) + b1_ref[...])
    w = _leaky(jnp.dot(bd2_ref[...], h.astype(jnp.bfloat16),
                       preferred_element_type=jnp.float32) + b2_ref[...])
    accs = []
    for m in range(M):
        am = jnp.zeros((CT, P), dtype=jnp.float32)
        for j in range(K):
            row = w[j * M + m:j * M + m + 1, :]
            am = am + jnp.broadcast_to(row, (CT, P)) * g_ref[j * CH:j * CH + CT, :]
        accs.append(am)
    acc = jnp.concatenate(accs, axis=0)
    out = jnp.dot(wl_ref[...], acc.astype(jnp.bfloat16),
                  preferred_element_type=jnp.float32)
    out_ref[0] = _leaky(out + blin_ref[...])


def kernel(xyz, features, knn_indices, w1, b1, w2, b2, Wlin, blin):
    B, _, N = xyz.shape
    Cin = features.shape[1]

    xyz_t = jnp.transpose(xyz, (0, 2, 1)).reshape(B * N, 3)
    feat_t = jnp.transpose(features, (0, 2, 1)).reshape(B * N, Cin)
    table = jnp.concatenate(
        [xyz_t, feat_t, jnp.zeros((B * N, CH - 3 - Cin), jnp.float32)], axis=1)

    idx = (knn_indices.astype(jnp.int32)
           + (jnp.arange(B, dtype=jnp.int32) * N)[:, None, None])
    idx = idx.reshape(B * N * K)

    gathered = _make_sc_gather(B * N * K)(table, idx)
    gt = jnp.transpose(gathered.reshape(B * N, K * CH))

    bd1 = jnp.kron(jnp.eye(K, dtype=jnp.float32), w1).astype(jnp.bfloat16)
    bd2 = jnp.kron(jnp.eye(K, dtype=jnp.float32), w2).astype(jnp.bfloat16)
    b1r = jnp.tile(b1, K).reshape(8 * K, 1)
    b2r = jnp.tile(b2, K).reshape(M * K, 1)

    wl = Wlin.reshape(COUT, M, 19)
    wl = jnp.concatenate(
        [wl, jnp.zeros((COUT, M, CT - 19), jnp.float32)], axis=2)
    wl = wl.reshape(COUT, M * CT).astype(jnp.bfloat16)

    nb = N // P
    out = pl.pallas_call(
        _tc_body,
        grid=(B, nb),
        in_specs=[
            pl.BlockSpec((K * CH, P), lambda b, i: (0, b * nb + i)),
            pl.BlockSpec((1, 3, P), lambda b, i: (b, 0, i)),
            pl.BlockSpec((8 * K, 3 * K), lambda b, i: (0, 0)),
            pl.BlockSpec((8 * K, 1), lambda b, i: (0, 0)),
            pl.BlockSpec((M * K, 8 * K), lambda b, i: (0, 0)),
            pl.BlockSpec((M * K, 1), lambda b, i: (0, 0)),
            pl.BlockSpec((COUT, M * CT), lambda b, i: (0, 0)),
            pl.BlockSpec((COUT, 1), lambda b, i: (0, 0)),
        ],
        out_specs=pl.BlockSpec((1, COUT, P), lambda b, i: (b, 0, i)),
        out_shape=jax.ShapeDtypeStruct((B, COUT, N), jnp.float32),
    )(gt, xyz, bd1, b1r, bd2, b2r, wl, blin.reshape(COUT, 1))

    return out


# speedup vs baseline: 1.2523x; 1.0367x over previous
import functools

import jax
import jax.numpy as jnp
from jax import lax
from jax.experimental import pallas as pl
from jax.experimental.pallas import tpu as pltpu
from jax.experimental.pallas import tpu_sc as plsc

K = 16
CH = 32
M = 16
CT = 24
COUT = 64
P = 1024
CHUNK = 512


def _leaky(x):
    return jnp.where(x >= 0, x, 0.1 * x)


def _make_sc_gather(rows):
    info = plsc.get_sparse_core_info()
    nc, ns = info.num_cores, info.num_subcores
    nw = nc * ns
    rows_per_w = rows // nw
    n2 = rows_per_w // (2 * CHUNK)
    mesh = plsc.VectorSubcoreMesh(core_axis_name="c", subcore_axis_name="s")

    @functools.partial(
        pl.kernel, mesh=mesh,
        compiler_params=pltpu.CompilerParams(use_tc_tiling_on_sc=False),
        out_type=jax.ShapeDtypeStruct((rows, CH), jnp.float32),
        scratch_types=[
            pltpu.VMEM((rows_per_w,), jnp.int32),
            pltpu.VMEM((CHUNK, CH), jnp.float32),
            pltpu.VMEM((CHUNK, CH), jnp.float32),
            pltpu.SemaphoreType.DMA,
            pltpu.SemaphoreType.DMA,
        ],
    )
    def sc_gather(table_hbm, idx_hbm, out_hbm, idx_v, rows_a, rows_b,
                  sem_a, sem_b):
        wid = lax.axis_index("s") * nc + lax.axis_index("c")
        base = wid * rows_per_w
        pltpu.sync_copy(idx_hbm.at[pl.ds(base, rows_per_w)], idx_v)

        def body(i, carry):
            c0 = 2 * i * CHUNK
            c1 = c0 + CHUNK
            ha = pltpu.async_copy(
                table_hbm.at[idx_v.at[pl.ds(c0, CHUNK)]], rows_a, sem_a)
            hb = pltpu.async_copy(
                table_hbm.at[idx_v.at[pl.ds(c1, CHUNK)]], rows_b, sem_b)
            ha.wait()
            pltpu.sync_copy(rows_a, out_hbm.at[pl.ds(base + c0, CHUNK)])
            hb.wait()
            pltpu.sync_copy(rows_b, out_hbm.at[pl.ds(base + c1, CHUNK)])
            return carry

        lax.fori_loop(0, n2, body, 0)

    return sc_gather


def _tc_body(g_ref, xyz_ref, bd1_ref, b1_ref, bd2_ref, b2_ref, wl_ref,
             blin_ref, out_ref):
    xyzb = xyz_ref[0]
    x48 = jnp.concatenate(
        [g_ref[j * CH:j * CH + 3, :] for j in range(K)], axis=0)
    xyzrep = jnp.concatenate([xyzb] * K, axis=0)
    dx = (x48 - xyzrep).astype(jnp.bfloat16)
    h = _leaky(jnp.dot(bd1_ref[...], dx,
                       preferred_element_type=jnp.float32) + b1_ref[...])
    w = _leaky(jnp.dot(bd2_ref[...], h.astype(jnp.bfloat16),
                       preferred_element_type=jnp.float32) + b2_ref[...])
    accs = []
    for m in range(M):
        am = jnp.zeros((CT, P), dtype=jnp.float32)
        for j in range(K):
            row = w[j * M + m:j * M + m + 1, :]
            am = am + jnp.broadcast_to(row, (CT, P)) * g_ref[j * CH:j * CH + CT, :]
        accs.append(am)
    acc = jnp.concatenate(accs, axis=0)
    out = jnp.dot(wl_ref[...], acc.astype(jnp.bfloat16),
                  preferred_element_type=jnp.float32)
    out_ref[0] = _leaky(out + blin_ref[...])


def kernel(xyz, features, knn_indices, w1, b1, w2, b2, Wlin, blin):
    B, _, N = xyz.shape
    Cin = features.shape[1]

    xyz_t = jnp.transpose(xyz, (0, 2, 1)).reshape(B * N, 3)
    feat_t = jnp.transpose(features, (0, 2, 1)).reshape(B * N, Cin)
    table = jnp.concatenate(
        [xyz_t, feat_t, jnp.zeros((B * N, CH - 3 - Cin), jnp.float32)], axis=1)

    idx = (knn_indices.astype(jnp.int32)
           + (jnp.arange(B, dtype=jnp.int32) * N)[:, None, None])
    idx = idx.reshape(B * N * K)

    gathered = _make_sc_gather(B * N * K)(table, idx)
    gt = jnp.transpose(gathered.reshape(B * N, K * CH))

    bd1 = jnp.kron(jnp.eye(K, dtype=jnp.float32), w1).astype(jnp.bfloat16)
    bd2 = jnp.kron(jnp.eye(K, dtype=jnp.float32), w2).astype(jnp.bfloat16)
    b1r = jnp.tile(b1, K).reshape(8 * K, 1)
    b2r = jnp.tile(b2, K).reshape(M * K, 1)

    wl = Wlin.reshape(COUT, M, 19)
    wl = jnp.concatenate(
        [wl, jnp.zeros((COUT, M, CT - 19), jnp.float32)], axis=2)
    wl = wl.reshape(COUT, M * CT).astype(jnp.bfloat16)

    nb = N // P
    out = pl.pallas_call(
        _tc_body,
        grid=(B, nb),
        in_specs=[
            pl.BlockSpec((K * CH, P), lambda b, i: (0, b * nb + i)),
            pl.BlockSpec((1, 3, P), lambda b, i: (b, 0, i)),
            pl.BlockSpec((8 * K, 3 * K), lambda b, i: (0, 0)),
            pl.BlockSpec((8 * K, 1), lambda b, i: (0, 0)),
            pl.BlockSpec((M * K, 8 * K), lambda b, i: (0, 0)),
            pl.BlockSpec((M * K, 1), lambda b, i: (0, 0)),
            pl.BlockSpec((COUT, M * CT), lambda b, i: (0, 0)),
            pl.BlockSpec((COUT, 1), lambda b, i: (0, 0)),
        ],
        out_specs=pl.BlockSpec((1, COUT, P), lambda b, i: (b, 0, i)),
        out_shape=jax.ShapeDtypeStruct((B, COUT, N), jnp.float32),
    )(gt, xyz, bd1, b1r, bd2, b2r, wl, blin.reshape(COUT, 1))

    return out
